# Initial kernel scaffold; baseline (speedup 1.0000x reference)
import jax
import jax.numpy as jnp
from jax.experimental import pallas as pl


def kernel(x, W_router):
    raise NotImplementedError("write your pallas kernel here")


# speedup vs baseline: 1.1594x; 1.1594x over previous
import functools

import jax
import jax.numpy as jnp
from jax import lax
from jax.experimental import pallas as pl
from jax.experimental.pallas import tpu as pltpu

D_MODEL = 2048
NUM_EXPERTS = 64
TOP_K = 8
N_TOKENS = 16384

BLK = 512


def _router_body(x_ref, w_ref, logits_ref, scores_ref, ew_ref, ei_ref, cnt_ref):
    logits = jnp.dot(x_ref[...], w_ref[...], preferred_element_type=jnp.float32)
    logits_ref[...] = logits
    m = jnp.max(logits, axis=-1, keepdims=True)
    e = jnp.exp(logits - m)
    s = e / jnp.sum(e, axis=-1, keepdims=True)
    scores_ref[...] = s

    iota = lax.broadcasted_iota(jnp.int32, s.shape, 1)
    work = s
    sel_any = jnp.zeros(s.shape, jnp.bool_)
    wcols = []
    icols = []
    for _ in range(TOP_K):
        mk = jnp.max(work, axis=-1, keepdims=True)
        idx = jnp.min(
            jnp.where(work == mk, iota, NUM_EXPERTS), axis=-1, keepdims=True
        )
        sel = iota == idx
        work = jnp.where(sel, -1.0, work)
        sel_any = jnp.logical_or(sel_any, sel)
        wcols.append(mk)
        icols.append(idx)
    ew_ref[...] = jnp.concatenate(wcols, axis=1)
    ei_ref[...] = jnp.concatenate(icols, axis=1)

    cnt = jnp.sum(sel_any.astype(jnp.int32), axis=0)[None, :]

    @pl.when(pl.program_id(0) == 0)
    def _():
        cnt_ref[...] = jnp.zeros_like(cnt_ref)

    cnt_ref[...] += cnt


@functools.partial(jax.jit, static_argnames=())
def kernel(x, W_router):
    n_blocks = N_TOKENS // BLK
    outs = pl.pallas_call(
        _router_body,
        grid=(n_blocks,),
        in_specs=[
            pl.BlockSpec((BLK, D_MODEL), lambda i: (i, 0)),
            pl.BlockSpec((D_MODEL, NUM_EXPERTS), lambda i: (0, 0)),
        ],
        out_specs=[
            pl.BlockSpec((BLK, NUM_EXPERTS), lambda i: (i, 0)),
            pl.BlockSpec((BLK, NUM_EXPERTS), lambda i: (i, 0)),
            pl.BlockSpec((BLK, TOP_K), lambda i: (i, 0)),
            pl.BlockSpec((BLK, TOP_K), lambda i: (i, 0)),
            pl.BlockSpec((1, NUM_EXPERTS), lambda i: (0, 0)),
        ],
        out_shape=[
            jax.ShapeDtypeStruct((N_TOKENS, NUM_EXPERTS), jnp.float32),
            jax.ShapeDtypeStruct((N_TOKENS, NUM_EXPERTS), jnp.float32),
            jax.ShapeDtypeStruct((N_TOKENS, TOP_K), jnp.float32),
            jax.ShapeDtypeStruct((N_TOKENS, TOP_K), jnp.int32),
            jax.ShapeDtypeStruct((1, NUM_EXPERTS), jnp.int32),
        ],
    )(x, W_router)
    logits, scores, ew, ei, cnt = outs
    return (logits, scores, ew, ei, cnt.reshape(NUM_EXPERTS))


# speedup vs baseline: 1.1857x; 1.0226x over previous
import functools

import jax
import jax.numpy as jnp
from jax import lax
from jax.experimental import pallas as pl
from jax.experimental.pallas import tpu as pltpu
from jax.experimental.pallas import tpu_sc as plsc

D_MODEL = 2048
NUM_EXPERTS = 64
TOP_K = 8
N_TOKENS = 16384

BLK = 512

_NC = 2
_NS = 16
_NW = _NC * _NS
_ROWS_PER_W = N_TOKENS // _NW
_L = 16


def _router_body(x_ref, w_ref, logits_ref, scores_ref):
    logits = jnp.dot(x_ref[...], w_ref[...], preferred_element_type=jnp.float32)
    logits_ref[...] = logits
    m = jnp.max(logits, axis=-1, keepdims=True)
    e = jnp.exp(logits - m)
    scores_ref[...] = e / jnp.sum(e, axis=-1, keepdims=True)


def _tc_router(x, W_router):
    n_blocks = N_TOKENS // BLK
    return pl.pallas_call(
        _router_body,
        grid=(n_blocks,),
        in_specs=[
            pl.BlockSpec((BLK, D_MODEL), lambda i: (i, 0)),
            pl.BlockSpec((D_MODEL, NUM_EXPERTS), lambda i: (0, 0)),
        ],
        out_specs=[
            pl.BlockSpec((BLK, NUM_EXPERTS), lambda i: (i, 0)),
            pl.BlockSpec((BLK, NUM_EXPERTS), lambda i: (i, 0)),
        ],
        out_shape=[
            jax.ShapeDtypeStruct((N_TOKENS, NUM_EXPERTS), jnp.float32),
            jax.ShapeDtypeStruct((N_TOKENS, NUM_EXPERTS), jnp.float32),
        ],
    )(x, W_router)


def _sc_topk_body(scores_hbm, ew_hbm, ei_hbm, cnt_hbm, s_v, ew_v, ei_v, hist_v, sem):
    c = lax.axis_index("c")
    s_id = lax.axis_index("s")
    wid = s_id * _NC + c
    base = wid * _ROWS_PER_W

    pltpu.sync_copy(scores_hbm.at[pl.ds(base, _ROWS_PER_W)], s_v)

    iota = lax.iota(jnp.int32, _L)
    lane_lt8 = iota < TOP_K
    zeros16 = jnp.zeros((_L,), jnp.int32)
    ones16 = jnp.ones((_L,), jnp.int32)
    for j in range(NUM_EXPERTS // _L):
        hist_v[pl.ds(_L * j, _L)] = zeros16

    def merge(ka, va, kb, vb):
        mk = jnp.where(lane_lt8, ka, lax.rev(kb, (0,)))
        mv = jnp.where(lane_lt8, va, lax.rev(vb, (0,)))
        return plsc.sort_key_val(mk, mv, descending=True)

    def row_body(r, carry):
        ks, vs = [], []
        for j in range(NUM_EXPERTS // _L):
            kj = s_v[r, pl.ds(_L * j, _L)]
            sk, sv = plsc.sort_key_val(kj, iota + _L * j, descending=True)
            ks.append(sk)
            vs.append(sv)
        k01, v01 = merge(ks[0], vs[0], ks[1], vs[1])
        k23, v23 = merge(ks[2], vs[2], ks[3], vs[3])
        kf, vf = merge(k01, v01, k23, v23)
        out_idx = r * TOP_K + iota
        plsc.store_scatter(ew_v, [out_idx], kf, mask=lane_lt8)
        plsc.store_scatter(ei_v, [out_idx], vf, mask=lane_lt8)
        plsc.addupdate_scatter(hist_v, [vf], ones16, mask=lane_lt8)
        return carry

    lax.fori_loop(0, _ROWS_PER_W, row_body, 0)

    pltpu.sync_copy(ew_v, ew_hbm.at[pl.ds(base * TOP_K, _ROWS_PER_W * TOP_K)])
    pltpu.sync_copy(ei_v, ei_hbm.at[pl.ds(base * TOP_K, _ROWS_PER_W * TOP_K)])
    pltpu.sync_copy(hist_v, cnt_hbm.at[wid])


@functools.cache
def _sc_topk():
    return pl.kernel(
        _sc_topk_body,
        out_type=[
            jax.ShapeDtypeStruct((N_TOKENS * TOP_K,), jnp.float32),
            jax.ShapeDtypeStruct((N_TOKENS * TOP_K,), jnp.int32),
            jax.ShapeDtypeStruct((_NW, NUM_EXPERTS), jnp.int32),
        ],
        mesh=plsc.VectorSubcoreMesh(
            core_axis_name="c", subcore_axis_name="s",
            num_cores=_NC, num_subcores=_NS,
        ),
        compiler_params=pltpu.CompilerParams(needs_layout_passes=False),
        scratch_types=[
            pltpu.VMEM((_ROWS_PER_W, NUM_EXPERTS), jnp.float32),
            pltpu.VMEM((_ROWS_PER_W * TOP_K,), jnp.float32),
            pltpu.VMEM((_ROWS_PER_W * TOP_K,), jnp.int32),
            pltpu.VMEM((NUM_EXPERTS,), jnp.int32),
            pltpu.SemaphoreType.DMA,
        ],
    )


def kernel(x, W_router):
    logits, scores = _tc_router(x, W_router)
    ew_flat, ei_flat, cnt_p = _sc_topk()(scores)
    ew = ew_flat.reshape(N_TOKENS, TOP_K)
    ei = ei_flat.reshape(N_TOKENS, TOP_K)
    cnt = jnp.sum(cnt_p, axis=0, dtype=jnp.int32)
    return (logits, scores, ew, ei, cnt)


# speedup vs baseline: 1.2869x; 1.0854x over previous
import functools

import jax
import jax.numpy as jnp
from jax import lax
from jax.experimental import pallas as pl
from jax.experimental.pallas import tpu as pltpu
from jax.experimental.pallas import tpu_sc as plsc

D_MODEL = 2048
NUM_EXPERTS = 64
TOP_K = 8
N_TOKENS = 16384

BLK = 512

_NC = 2
_NS = 16
_NW = _NC * _NS
_ROWS_PER_W = N_TOKENS // _NW
_L = 16


def _router_body(x_ref, w_ref, logits_ref, scores_ref):
    logits = jnp.dot(x_ref[...], w_ref[...], preferred_element_type=jnp.float32)
    logits_ref[...] = logits
    m = jnp.max(logits, axis=-1, keepdims=True)
    e = jnp.exp(logits - m)
    scores_ref[...] = e / jnp.sum(e, axis=-1, keepdims=True)


def _tc_router(x, W_router):
    n_blocks = N_TOKENS // BLK
    return pl.pallas_call(
        _router_body,
        grid=(n_blocks,),
        in_specs=[
            pl.BlockSpec((BLK, D_MODEL), lambda i: (i, 0)),
            pl.BlockSpec((D_MODEL, NUM_EXPERTS), lambda i: (0, 0)),
        ],
        out_specs=[
            pl.BlockSpec((BLK, NUM_EXPERTS), lambda i: (i, 0)),
            pl.BlockSpec((BLK, NUM_EXPERTS), lambda i: (i, 0)),
        ],
        out_shape=[
            jax.ShapeDtypeStruct((N_TOKENS, NUM_EXPERTS), jnp.float32),
            jax.ShapeDtypeStruct((N_TOKENS, NUM_EXPERTS), jnp.float32),
        ],
    )(x, W_router)


def _sc_topk_body(scores_hbm, ew_hbm, ei_hbm, cnt_hbm, s_v, ew_v, ei_v, hist_v, sem):
    c = lax.axis_index("c")
    s_id = lax.axis_index("s")
    wid = s_id * _NC + c
    base = wid * _ROWS_PER_W

    pltpu.sync_copy(scores_hbm.at[pl.ds(base, _ROWS_PER_W)], s_v)

    iota = lax.iota(jnp.int32, _L)
    lane_lt8 = iota < TOP_K
    zeros16 = jnp.zeros((_L,), jnp.int32)
    ones16 = jnp.ones((_L,), jnp.int32)
    for j in range(NUM_EXPERTS // _L):
        hist_v[pl.ds(_L * j, _L)] = zeros16

    def merge(ka, va, kb, vb):
        mk = jnp.where(lane_lt8, ka, lax.rev(kb, (0,)))
        mv = jnp.where(lane_lt8, va, lax.rev(vb, (0,)))
        return plsc.sort_key_val(mk, mv, descending=True)

    @plsc.parallel_loop(0, _ROWS_PER_W, 1, unroll=8)
    def row_body(r):
        ks, vs = [], []
        for j in range(NUM_EXPERTS // _L):
            kj = s_v[r, pl.ds(_L * j, _L)]
            sk, sv = plsc.sort_key_val(kj, iota + _L * j, descending=True)
            ks.append(sk)
            vs.append(sv)
        k01, v01 = merge(ks[0], vs[0], ks[1], vs[1])
        k23, v23 = merge(ks[2], vs[2], ks[3], vs[3])
        kf, vf = merge(k01, v01, k23, v23)
        out_idx = r * TOP_K + iota
        plsc.store_scatter(ew_v, [out_idx], kf, mask=lane_lt8)
        plsc.store_scatter(ei_v, [out_idx], vf, mask=lane_lt8)

    def hist_body(i, carry):
        r16 = i * (4 * _L)
        for u in range(4):
            v = ei_v[pl.ds(r16 + u * _L, _L)]
            plsc.addupdate_scatter(hist_v, [v], ones16)
        return carry

    lax.fori_loop(0, _ROWS_PER_W * TOP_K // (4 * _L), hist_body, 0)

    pltpu.sync_copy(ew_v, ew_hbm.at[pl.ds(base * TOP_K, _ROWS_PER_W * TOP_K)])
    pltpu.sync_copy(ei_v, ei_hbm.at[pl.ds(base * TOP_K, _ROWS_PER_W * TOP_K)])
    pltpu.sync_copy(hist_v, cnt_hbm.at[wid])


@functools.cache
def _sc_topk():
    return pl.kernel(
        _sc_topk_body,
        out_type=[
            jax.ShapeDtypeStruct((N_TOKENS * TOP_K,), jnp.float32),
            jax.ShapeDtypeStruct((N_TOKENS * TOP_K,), jnp.int32),
            jax.ShapeDtypeStruct((_NW, NUM_EXPERTS), jnp.int32),
        ],
        mesh=plsc.VectorSubcoreMesh(
            core_axis_name="c", subcore_axis_name="s",
            num_cores=_NC, num_subcores=_NS,
        ),
        compiler_params=pltpu.CompilerParams(needs_layout_passes=False),
        scratch_types=[
            pltpu.VMEM((_ROWS_PER_W, NUM_EXPERTS), jnp.float32),
            pltpu.VMEM((_ROWS_PER_W * TOP_K,), jnp.float32),
            pltpu.VMEM((_ROWS_PER_W * TOP_K,), jnp.int32),
            pltpu.VMEM((NUM_EXPERTS,), jnp.int32),
            pltpu.SemaphoreType.DMA,
        ],
    )


def kernel(x, W_router):
    logits, scores = _tc_router(x, W_router)
    ew_flat, ei_flat, cnt_p = _sc_topk()(scores)
    ew = ew_flat.reshape(N_TOKENS, TOP_K)
    ei = ei_flat.reshape(N_TOKENS, TOP_K)
    cnt = jnp.sum(cnt_p, axis=0, dtype=jnp.int32)
    return (logits, scores, ew, ei, cnt)
